# baseline (speedup 1.0000x reference)
import functools

import jax
import jax.numpy as jnp
from jax.experimental import pallas as pl
from jax.experimental.pallas import tpu as pltpu

N = 10000
E = 160000
D_IN = 128
D_EDGE = 16
D_OUT = 16
EH = 128
STEPS = 3
B = 256
FFN_H = 300
N_TASKS = 8


_NBLK = 2000
_EBLK = 4000


def _proj_body(x_ref, w_ref, b_ref, o_ref):
    o_ref[...] = jax.nn.relu(
        jnp.dot(x_ref[...], w_ref[...], preferred_element_type=jnp.float32, precision=jax.lax.Precision.HIGHEST)
        + b_ref[...]
    )


def _proj(x, w, b):
    return pl.pallas_call(
        _proj_body,
        grid=(N // _NBLK,),
        in_specs=[
            pl.BlockSpec((_NBLK, D_IN), lambda i: (i, 0)),
            pl.BlockSpec((D_IN, D_OUT), lambda i: (0, 0)),
            pl.BlockSpec((1, D_OUT), lambda i: (0, 0)),
        ],
        out_specs=pl.BlockSpec((_NBLK, D_OUT), lambda i: (i, 0)),
        out_shape=jax.ShapeDtypeStruct((N, D_OUT), jnp.float32),
    )(x, w, b.reshape(1, D_OUT))


def _msg_body(ea_ref, hs_ref, w1_ref, b1_ref, w2_ref, b2_ref, o_ref):
    a = jax.nn.relu(
        jnp.dot(ea_ref[...], w1_ref[...], preferred_element_type=jnp.float32, precision=jax.lax.Precision.HIGHEST)
        + b1_ref[...]
    )
    w = jnp.dot(a, w2_ref[...], preferred_element_type=jnp.float32, precision=jax.lax.Precision.HIGHEST) + b2_ref[...]
    h = hs_ref[...]
    m = h[:, 0:1] * w[:, 0:D_OUT]
    for d in range(1, D_OUT):
        m = m + h[:, d : d + 1] * w[:, d * D_OUT : (d + 1) * D_OUT]
    o_ref[...] = m


def _msg(edge_attr, h_src, w1, b1, w2, b2):
    return pl.pallas_call(
        _msg_body,
        grid=(E // _EBLK,),
        in_specs=[
            pl.BlockSpec((_EBLK, D_EDGE), lambda i: (i, 0)),
            pl.BlockSpec((_EBLK, D_OUT), lambda i: (i, 0)),
            pl.BlockSpec((D_EDGE, EH), lambda i: (0, 0)),
            pl.BlockSpec((1, EH), lambda i: (0, 0)),
            pl.BlockSpec((EH, D_OUT * D_OUT), lambda i: (0, 0)),
            pl.BlockSpec((1, D_OUT * D_OUT), lambda i: (0, 0)),
        ],
        out_specs=pl.BlockSpec((_EBLK, D_OUT), lambda i: (i, 0)),
        out_shape=jax.ShapeDtypeStruct((E, D_OUT), jnp.float32),
    )(edge_attr, h_src, w1, b1.reshape(1, EH), w2, b2.reshape(1, D_OUT * D_OUT))


def _gru_body(p0_ref, p1_ref, cb_ref, h_ref, wihT_ref, whhT_ref, bih_ref,
              bhh_ref, o_ref):
    nf = jax.nn.relu(p0_ref[...] + p1_ref[...] + cb_ref[...])
    h = h_ref[...]
    gi = jnp.dot(nf, wihT_ref[...], preferred_element_type=jnp.float32, precision=jax.lax.Precision.HIGHEST) + bih_ref[...]
    gh = jnp.dot(h, whhT_ref[...], preferred_element_type=jnp.float32, precision=jax.lax.Precision.HIGHEST) + bhh_ref[...]
    i_r, i_z, i_n = gi[:, :D_OUT], gi[:, D_OUT:2 * D_OUT], gi[:, 2 * D_OUT:]
    h_r, h_z, h_n = gh[:, :D_OUT], gh[:, D_OUT:2 * D_OUT], gh[:, 2 * D_OUT:]
    r = jax.nn.sigmoid(i_r + h_r)
    z = jax.nn.sigmoid(i_z + h_z)
    n = jnp.tanh(i_n + r * h_n)
    o_ref[...] = (1.0 - z) * n + z * h


def _gru(p0, p1, conv_b, h, wihT, whhT, bih, bhh):
    g3 = 3 * D_OUT
    return pl.pallas_call(
        _gru_body,
        grid=(N // _NBLK,),
        in_specs=[
            pl.BlockSpec((_NBLK, D_OUT), lambda i: (i, 0)),
            pl.BlockSpec((_NBLK, D_OUT), lambda i: (i, 0)),
            pl.BlockSpec((1, D_OUT), lambda i: (0, 0)),
            pl.BlockSpec((_NBLK, D_OUT), lambda i: (i, 0)),
            pl.BlockSpec((D_OUT, g3), lambda i: (0, 0)),
            pl.BlockSpec((D_OUT, g3), lambda i: (0, 0)),
            pl.BlockSpec((1, g3), lambda i: (0, 0)),
            pl.BlockSpec((1, g3), lambda i: (0, 0)),
        ],
        out_specs=pl.BlockSpec((_NBLK, D_OUT), lambda i: (i, 0)),
        out_shape=jax.ShapeDtypeStruct((N, D_OUT), jnp.float32),
    )(p0, p1, conv_b.reshape(1, D_OUT), h, wihT, whhT,
      bih.reshape(1, g3), bhh.reshape(1, g3))


def _ffn_body(mol_ref, w1_ref, b1_ref, w2_ref, b2_ref, w3_ref, b3_ref, o_ref):
    h1 = jax.nn.relu(
        jnp.dot(mol_ref[...], w1_ref[...], preferred_element_type=jnp.float32, precision=jax.lax.Precision.HIGHEST)
        + b1_ref[...]
    )
    h2 = jax.nn.relu(
        jnp.dot(h1, w2_ref[...], preferred_element_type=jnp.float32, precision=jax.lax.Precision.HIGHEST) + b2_ref[...]
    )
    o_ref[...] = (
        jnp.dot(h2, w3_ref[...], preferred_element_type=jnp.float32, precision=jax.lax.Precision.HIGHEST) + b3_ref[...]
    )


def _ffn(mol, w1, b1, w2, b2, w3, b3):
    din = D_OUT + D_EDGE
    return pl.pallas_call(
        _ffn_body,
        in_specs=[
            pl.BlockSpec((B, din), lambda: (0, 0)),
            pl.BlockSpec((din, FFN_H), lambda: (0, 0)),
            pl.BlockSpec((1, FFN_H), lambda: (0, 0)),
            pl.BlockSpec((FFN_H, FFN_H), lambda: (0, 0)),
            pl.BlockSpec((1, FFN_H), lambda: (0, 0)),
            pl.BlockSpec((FFN_H, N_TASKS), lambda: (0, 0)),
            pl.BlockSpec((1, N_TASKS), lambda: (0, 0)),
        ],
        out_specs=pl.BlockSpec((B, N_TASKS), lambda: (0, 0)),
        out_shape=jax.ShapeDtypeStruct((B, N_TASKS), jnp.float32),
    )(mol, w1, b1.reshape(1, FFN_H), w2, b2.reshape(1, FFN_H), w3,
      b3.reshape(1, N_TASKS))


def kernel(x, edge_index, edge_attr, node_graph_ids, proj_W, proj_b, en_W1,
           en_b1, en_W2, en_b2, conv_b, gru_Wih, gru_Whh, gru_bih, gru_bhh,
           f1_W, f1_b, f2_W, f2_b, f3_W, f3_b):
    src = edge_index[0]
    dst = edge_index[1]
    wihT = gru_Wih.T
    whhT = gru_Whh.T

    h = _proj(x, proj_W, proj_b)

    zero_n = jnp.zeros((N, D_OUT), jnp.float32)
    for _ in range(STEPS):
        h_src = h[src]
        m = _msg(edge_attr, h_src, en_W1, en_b1, en_W2, en_b2)
        agg = zero_n.at[dst].add(m)
        h = _gru(agg, zero_n, conv_b, h, wihT, whhT, gru_bih, gru_bhh)

    hs = h[src]
    gid_dst = node_graph_ids[dst]
    molH = jnp.zeros((B, D_OUT), jnp.float32).at[gid_dst].add(hs)
    molE = jnp.zeros((B, D_EDGE), jnp.float32).at[gid_dst].add(edge_attr)
    mol = jnp.concatenate([molH, molE], axis=1)
    return _ffn(mol, f1_W, f1_b, f2_W, f2_b, f3_W, f3_b)


# speedup vs baseline: 1.8186x; 1.8186x over previous
import functools

import jax
import jax.numpy as jnp
from jax import lax
from jax.experimental import pallas as pl
from jax.experimental.pallas import tpu as pltpu
from jax.experimental.pallas import tpu_sc as plsc

N = 10000
E = 160000
D_IN = 128
D_EDGE = 16
D_OUT = 16
EH = 128
STEPS = 3
B = 256
FFN_H = 300
N_TASKS = 8

_NC = 2
_NS = 16
_NW = _NC * _NS
_CHUNK = 128
_EP = 163840
_EPW = _EP // _NW
_NCH = _EPW // _CHUNK
_NP = 10240
_ZROWS = _NP // _NS


def _sc_mesh():
    return plsc.VectorSubcoreMesh(
        core_axis_name="c", subcore_axis_name="s",
        num_cores=_NC, num_subcores=_NS)


_SC_PARAMS = pltpu.CompilerParams(use_tc_tiling_on_sc=False)


def _sc_gather(table, idx3):
    @functools.partial(
        pl.kernel,
        mesh=_sc_mesh(),
        out_type=jax.ShapeDtypeStruct((_EP, D_OUT), jnp.float32),
        scratch_types=[
            pltpu.VMEM((_NCH, _CHUNK), jnp.int32),
            pltpu.VMEM((_EPW, D_OUT), jnp.float32),
            pltpu.SemaphoreType.DMA,
        ],
        compiler_params=_SC_PARAMS,
    )
    def k(table_hbm, idx_hbm, out_hbm, idx_v, rows_v, sem):
        wid = lax.axis_index("s") * _NC + lax.axis_index("c")
        pltpu.sync_copy(idx_hbm.at[wid], idx_v)
        copies = []
        for j in range(_NCH):
            copies.append(
                pltpu.async_copy(
                    table_hbm.at[idx_v.at[j]],
                    rows_v.at[pl.ds(j * _CHUNK, _CHUNK)],
                    sem,
                )
            )
        for cp in copies:
            cp.wait()
        pltpu.sync_copy(rows_v, out_hbm.at[pl.ds(wid * _EPW, _EPW)])

    return k(table, idx3)


def _sc_scatter_add(vals, idx3):
    @functools.partial(
        pl.kernel,
        mesh=_sc_mesh(),
        out_type=jax.ShapeDtypeStruct((_NC, _NP, D_OUT), jnp.float32),
        scratch_types=[
            pltpu.VMEM((_NCH, _CHUNK), jnp.int32),
            pltpu.VMEM((_EPW, D_OUT), jnp.float32),
            pltpu.VMEM((_CHUNK, D_OUT), jnp.float32),
            pltpu.VMEM_SHARED((_NP, D_OUT), jnp.float32),
            pltpu.SemaphoreType.DMA,
        ],
        compiler_params=_SC_PARAMS,
    )
    def k(vals_hbm, idx_hbm, out_hbm, idx_v, vals_v, stage_v, acc_sh, sem):
        cid = lax.axis_index("c")
        sid = lax.axis_index("s")
        wid = sid * _NC + cid

        def _zero_row(i, _):
            stage_v[i] = jnp.zeros((D_OUT,), jnp.float32)
            return 0

        lax.fori_loop(0, _CHUNK, _zero_row, 0)
        for t in range(_ZROWS // _CHUNK):
            pltpu.sync_copy(
                stage_v, acc_sh.at[pl.ds(sid * _ZROWS + t * _CHUNK, _CHUNK)]
            )
        plsc.subcore_barrier()

        pltpu.sync_copy(idx_hbm.at[wid], idx_v)
        pltpu.sync_copy(vals_hbm.at[pl.ds(wid * _EPW, _EPW)], vals_v)
        for j in range(_NCH):
            pltpu.sync_copy(
                vals_v.at[pl.ds(j * _CHUNK, _CHUNK)],
                acc_sh.at[idx_v.at[j]],
                add=True,
            )
        plsc.subcore_barrier()

        for t in range(_ZROWS // _CHUNK):
            r0 = sid * _ZROWS + t * _CHUNK
            pltpu.sync_copy(acc_sh.at[pl.ds(r0, _CHUNK)], stage_v)
            pltpu.sync_copy(stage_v, out_hbm.at[cid].at[pl.ds(r0, _CHUNK)])

    return k(vals, idx3)


_NBLK = 2000
_EBLK = 4096


def _proj_body(x_ref, w_ref, b_ref, o_ref):
    o_ref[...] = jax.nn.relu(
        jnp.dot(x_ref[...], w_ref[...], preferred_element_type=jnp.float32, precision=jax.lax.Precision.HIGHEST)
        + b_ref[...]
    )


def _proj(x, w, b):
    return pl.pallas_call(
        _proj_body,
        grid=(N // _NBLK,),
        in_specs=[
            pl.BlockSpec((_NBLK, D_IN), lambda i: (i, 0)),
            pl.BlockSpec((D_IN, D_OUT), lambda i: (0, 0)),
            pl.BlockSpec((1, D_OUT), lambda i: (0, 0)),
        ],
        out_specs=pl.BlockSpec((_NBLK, D_OUT), lambda i: (i, 0)),
        out_shape=jax.ShapeDtypeStruct((N, D_OUT), jnp.float32),
    )(x, w, b.reshape(1, D_OUT))


def _msg_body(ea_ref, hs_ref, w1_ref, b1_ref, w2_ref, b2_ref, o_ref):
    a = jax.nn.relu(
        jnp.dot(ea_ref[...], w1_ref[...], preferred_element_type=jnp.float32, precision=jax.lax.Precision.HIGHEST)
        + b1_ref[...]
    )
    w = jnp.dot(a, w2_ref[...], preferred_element_type=jnp.float32, precision=jax.lax.Precision.HIGHEST) + b2_ref[...]
    h = hs_ref[...]
    m = h[:, 0:1] * w[:, 0:D_OUT]
    for d in range(1, D_OUT):
        m = m + h[:, d : d + 1] * w[:, d * D_OUT : (d + 1) * D_OUT]
    o_ref[...] = m


def _msg(edge_attr, h_src, w1, b1, w2, b2):
    return pl.pallas_call(
        _msg_body,
        grid=(_EP // _EBLK,),
        in_specs=[
            pl.BlockSpec((_EBLK, D_EDGE), lambda i: (i, 0)),
            pl.BlockSpec((_EBLK, D_OUT), lambda i: (i, 0)),
            pl.BlockSpec((D_EDGE, EH), lambda i: (0, 0)),
            pl.BlockSpec((1, EH), lambda i: (0, 0)),
            pl.BlockSpec((EH, D_OUT * D_OUT), lambda i: (0, 0)),
            pl.BlockSpec((1, D_OUT * D_OUT), lambda i: (0, 0)),
        ],
        out_specs=pl.BlockSpec((_EBLK, D_OUT), lambda i: (i, 0)),
        out_shape=jax.ShapeDtypeStruct((_EP, D_OUT), jnp.float32),
    )(edge_attr, h_src, w1, b1.reshape(1, EH), w2, b2.reshape(1, D_OUT * D_OUT))


def _segsum_body(gid_ref, nsh_ref, nse_ref, oh_ref, oe_ref):
    i = pl.program_id(0)

    @pl.when(i == 0)
    def _init():
        oh_ref[...] = jnp.zeros_like(oh_ref)
        oe_ref[...] = jnp.zeros_like(oe_ref)

    gid = gid_ref[0, 0, :]
    sel = (jax.lax.broadcasted_iota(jnp.int32, (B, _NBLK), 0)
           == gid[None, :]).astype(jnp.float32)
    oh_ref[...] += jnp.dot(sel, nsh_ref[...],
                           preferred_element_type=jnp.float32,
                           precision=jax.lax.Precision.HIGHEST)
    oe_ref[...] += jnp.dot(sel, nse_ref[...],
                           preferred_element_type=jnp.float32,
                           precision=jax.lax.Precision.HIGHEST)


def _segsum(gid, nsh, nse):
    return pl.pallas_call(
        _segsum_body,
        grid=(N // _NBLK,),
        in_specs=[
            pl.BlockSpec((1, 1, _NBLK), lambda i: (i, 0, 0)),
            pl.BlockSpec((_NBLK, D_OUT), lambda i: (i, 0)),
            pl.BlockSpec((_NBLK, D_EDGE), lambda i: (i, 0)),
        ],
        out_specs=[
            pl.BlockSpec((B, D_OUT), lambda i: (0, 0)),
            pl.BlockSpec((B, D_EDGE), lambda i: (0, 0)),
        ],
        out_shape=[
            jax.ShapeDtypeStruct((B, D_OUT), jnp.float32),
            jax.ShapeDtypeStruct((B, D_EDGE), jnp.float32),
        ],
    )(gid.reshape(N // _NBLK, 1, _NBLK), nsh, nse)


def _gru_body(p0_ref, p1_ref, cb_ref, h_ref, wihT_ref, whhT_ref, bih_ref,
              bhh_ref, o_ref):
    nf = jax.nn.relu(p0_ref[...] + p1_ref[...] + cb_ref[...])
    h = h_ref[...]
    gi = jnp.dot(nf, wihT_ref[...], preferred_element_type=jnp.float32, precision=jax.lax.Precision.HIGHEST) + bih_ref[...]
    gh = jnp.dot(h, whhT_ref[...], preferred_element_type=jnp.float32, precision=jax.lax.Precision.HIGHEST) + bhh_ref[...]
    i_r, i_z, i_n = gi[:, :D_OUT], gi[:, D_OUT:2 * D_OUT], gi[:, 2 * D_OUT:]
    h_r, h_z, h_n = gh[:, :D_OUT], gh[:, D_OUT:2 * D_OUT], gh[:, 2 * D_OUT:]
    r = jax.nn.sigmoid(i_r + h_r)
    z = jax.nn.sigmoid(i_z + h_z)
    n = jnp.tanh(i_n + r * h_n)
    o_ref[...] = (1.0 - z) * n + z * h


def _gru(p0, p1, conv_b, h, wihT, whhT, bih, bhh):
    g3 = 3 * D_OUT
    return pl.pallas_call(
        _gru_body,
        grid=(N // _NBLK,),
        in_specs=[
            pl.BlockSpec((_NBLK, D_OUT), lambda i: (i, 0)),
            pl.BlockSpec((_NBLK, D_OUT), lambda i: (i, 0)),
            pl.BlockSpec((1, D_OUT), lambda i: (0, 0)),
            pl.BlockSpec((_NBLK, D_OUT), lambda i: (i, 0)),
            pl.BlockSpec((D_OUT, g3), lambda i: (0, 0)),
            pl.BlockSpec((D_OUT, g3), lambda i: (0, 0)),
            pl.BlockSpec((1, g3), lambda i: (0, 0)),
            pl.BlockSpec((1, g3), lambda i: (0, 0)),
        ],
        out_specs=pl.BlockSpec((_NBLK, D_OUT), lambda i: (i, 0)),
        out_shape=jax.ShapeDtypeStruct((N, D_OUT), jnp.float32),
    )(p0, p1, conv_b.reshape(1, D_OUT), h, wihT, whhT,
      bih.reshape(1, g3), bhh.reshape(1, g3))


def _ffn_body(mol_ref, w1_ref, b1_ref, w2_ref, b2_ref, w3_ref, b3_ref, o_ref):
    h1 = jax.nn.relu(
        jnp.dot(mol_ref[...], w1_ref[...], preferred_element_type=jnp.float32, precision=jax.lax.Precision.HIGHEST)
        + b1_ref[...]
    )
    h2 = jax.nn.relu(
        jnp.dot(h1, w2_ref[...], preferred_element_type=jnp.float32, precision=jax.lax.Precision.HIGHEST) + b2_ref[...]
    )
    o_ref[...] = (
        jnp.dot(h2, w3_ref[...], preferred_element_type=jnp.float32, precision=jax.lax.Precision.HIGHEST) + b3_ref[...]
    )


def _ffn(mol, w1, b1, w2, b2, w3, b3):
    din = D_OUT + D_EDGE
    return pl.pallas_call(
        _ffn_body,
        in_specs=[
            pl.BlockSpec((B, din), lambda: (0, 0)),
            pl.BlockSpec((din, FFN_H), lambda: (0, 0)),
            pl.BlockSpec((1, FFN_H), lambda: (0, 0)),
            pl.BlockSpec((FFN_H, FFN_H), lambda: (0, 0)),
            pl.BlockSpec((1, FFN_H), lambda: (0, 0)),
            pl.BlockSpec((FFN_H, N_TASKS), lambda: (0, 0)),
            pl.BlockSpec((1, N_TASKS), lambda: (0, 0)),
        ],
        out_specs=pl.BlockSpec((B, N_TASKS), lambda: (0, 0)),
        out_shape=jax.ShapeDtypeStruct((B, N_TASKS), jnp.float32),
    )(mol, w1, b1.reshape(1, FFN_H), w2, b2.reshape(1, FFN_H), w3,
      b3.reshape(1, N_TASKS))


def kernel(x, edge_index, edge_attr, node_graph_ids, proj_W, proj_b, en_W1,
           en_b1, en_W2, en_b2, conv_b, gru_Wih, gru_Whh, gru_bih, gru_bhh,
           f1_W, f1_b, f2_W, f2_b, f3_W, f3_b):
    src = edge_index[0]
    dst = edge_index[1]
    wihT = gru_Wih.T
    whhT = gru_Whh.T

    pad = _EP - E
    src3 = jnp.concatenate(
        [src, jnp.zeros((pad,), jnp.int32)]).reshape(_NW, _NCH, _CHUNK)
    dst3 = jnp.concatenate(
        [dst, jnp.full((pad,), N, jnp.int32)]).reshape(_NW, _NCH, _CHUNK)
    ea_p = jnp.pad(edge_attr, ((0, pad), (0, 0)))

    h = _proj(x, proj_W, proj_b)

    for _ in range(STEPS):
        h_src = _sc_gather(h, src3)
        m = _msg(ea_p, h_src, en_W1, en_b1, en_W2, en_b2)
        parts = _sc_scatter_add(m, dst3)
        h = _gru(parts[0, :N], parts[1, :N], conv_b, h, wihT, whhT,
                 gru_bih, gru_bhh)

    hs = _sc_gather(h, src3)
    ph = _sc_scatter_add(hs, dst3)
    pe = _sc_scatter_add(ea_p, dst3)
    molH, molE = _segsum(node_graph_ids,
                         ph[0, :N] + ph[1, :N], pe[0, :N] + pe[1, :N])
    mol = jnp.concatenate([molH, molE], axis=1)
    return _ffn(mol, f1_W, f1_b, f2_W, f2_b, f3_W, f3_b)


# speedup vs baseline: 4.0997x; 2.2543x over previous
import functools

import jax
import jax.numpy as jnp
from jax import lax
from jax.experimental import pallas as pl
from jax.experimental.pallas import tpu as pltpu
from jax.experimental.pallas import tpu_sc as plsc

N = 10000
E = 160000
D_IN = 128
D_EDGE = 16
D_OUT = 16
EH = 128
STEPS = 3
B = 256
FFN_H = 300
N_TASKS = 8

_NC = 2
_NS = 16
_NW = _NC * _NS
_CHUNK = 128
_EP = 163840
_EPW = _EP // _NW
_NCH = _EPW // _CHUNK
_NP = 10240
_ZROWS = _NP // _NS


def _sc_mesh():
    return plsc.VectorSubcoreMesh(
        core_axis_name="c", subcore_axis_name="s",
        num_cores=_NC, num_subcores=_NS)


_SC_PARAMS = pltpu.CompilerParams(use_tc_tiling_on_sc=False)


def _sc_gather(table, idx3):
    @functools.partial(
        pl.kernel,
        mesh=_sc_mesh(),
        out_type=jax.ShapeDtypeStruct((_EP, D_OUT), jnp.float32),
        scratch_types=[
            pltpu.VMEM((_NCH, _CHUNK), jnp.int32),
            pltpu.VMEM((_EPW, D_OUT), jnp.float32),
            pltpu.SemaphoreType.DMA,
        ],
        compiler_params=_SC_PARAMS,
    )
    def k(table_hbm, idx_hbm, out_hbm, idx_v, rows_v, sem):
        wid = lax.axis_index("s") * _NC + lax.axis_index("c")
        pltpu.sync_copy(idx_hbm.at[wid], idx_v)
        copies = []
        for j in range(_NCH):
            copies.append(
                pltpu.async_copy(
                    table_hbm.at[idx_v.at[j]],
                    rows_v.at[pl.ds(j * _CHUNK, _CHUNK)],
                    sem,
                )
            )
        for cp in copies:
            cp.wait()
        pltpu.sync_copy(rows_v, out_hbm.at[pl.ds(wid * _EPW, _EPW)])

    return k(table, idx3)


def _sc_scatter_add(vals, idx3):
    @functools.partial(
        pl.kernel,
        mesh=_sc_mesh(),
        out_type=jax.ShapeDtypeStruct((_NC, _NP, D_OUT), jnp.float32),
        scratch_types=[
            pltpu.VMEM((_NCH, _CHUNK), jnp.int32),
            pltpu.VMEM((_EPW, D_OUT), jnp.float32),
            pltpu.VMEM((_CHUNK, D_OUT), jnp.float32),
            pltpu.VMEM_SHARED((_NP, D_OUT), jnp.float32),
            pltpu.SemaphoreType.DMA,
        ],
        compiler_params=_SC_PARAMS,
    )
    def k(vals_hbm, idx_hbm, out_hbm, idx_v, vals_v, stage_v, acc_sh, sem):
        cid = lax.axis_index("c")
        sid = lax.axis_index("s")
        wid = sid * _NC + cid

        def _zero_row(i, _):
            stage_v[i] = jnp.zeros((D_OUT,), jnp.float32)
            return 0

        lax.fori_loop(0, _CHUNK, _zero_row, 0)
        for t in range(_ZROWS // _CHUNK):
            pltpu.sync_copy(
                stage_v, acc_sh.at[pl.ds(sid * _ZROWS + t * _CHUNK, _CHUNK)]
            )
        plsc.subcore_barrier()

        pltpu.sync_copy(idx_hbm.at[wid], idx_v)
        pltpu.sync_copy(vals_hbm.at[pl.ds(wid * _EPW, _EPW)], vals_v)
        for j in range(_NCH):
            pltpu.sync_copy(
                vals_v.at[pl.ds(j * _CHUNK, _CHUNK)],
                acc_sh.at[idx_v.at[j]],
                add=True,
            )
        plsc.subcore_barrier()

        for t in range(_ZROWS // _CHUNK):
            r0 = sid * _ZROWS + t * _CHUNK
            pltpu.sync_copy(acc_sh.at[pl.ds(r0, _CHUNK)], stage_v)
            pltpu.sync_copy(stage_v, out_hbm.at[cid].at[pl.ds(r0, _CHUNK)])

    return k(vals, idx3)


_NBLK = 2000
_EBLK = 4096


def _proj_body(x_ref, w_ref, b_ref, o_ref):
    o_ref[...] = jax.nn.relu(
        jnp.dot(x_ref[...], w_ref[...], preferred_element_type=jnp.float32, precision=jax.lax.Precision.HIGHEST)
        + b_ref[...]
    )


def _proj(x, w, b):
    return pl.pallas_call(
        _proj_body,
        grid=(N // _NBLK,),
        in_specs=[
            pl.BlockSpec((_NBLK, D_IN), lambda i: (i, 0)),
            pl.BlockSpec((D_IN, D_OUT), lambda i: (0, 0)),
            pl.BlockSpec((1, D_OUT), lambda i: (0, 0)),
        ],
        out_specs=pl.BlockSpec((_NBLK, D_OUT), lambda i: (i, 0)),
        out_shape=jax.ShapeDtypeStruct((N, D_OUT), jnp.float32),
    )(x, w, b.reshape(1, D_OUT))


def _dot3(x, w_hi, w_lo):
    x_hi = x.astype(jnp.bfloat16)
    x_lo = (x - x_hi.astype(jnp.float32)).astype(jnp.bfloat16)
    return (jnp.dot(x_hi, w_hi, preferred_element_type=jnp.float32)
            + (jnp.dot(x_hi, w_lo, preferred_element_type=jnp.float32)
               + jnp.dot(x_lo, w_hi, preferred_element_type=jnp.float32)))


def _msg_body(ea_ref, hs_ref, w1h_ref, w1l_ref, b1_ref, w2h_ref, w2l_ref,
              b2_ref, o_ref):
    a = jax.nn.relu(
        _dot3(ea_ref[...], w1h_ref[...], w1l_ref[...]) + b1_ref[...]
    )
    w = _dot3(a, w2h_ref[...], w2l_ref[...]) + b2_ref[...]
    lane = jax.lax.broadcasted_iota(
        jnp.int32, (_EBLK, D_OUT * D_OUT), 1) // D_OUT
    g = jnp.take_along_axis(hs_ref[...], lane, axis=1) * w
    r = g[:, :128] + g[:, 128:]
    r = r[:, :64] + r[:, 64:]
    r = r[:, :32] + r[:, 32:]
    o_ref[...] = r[:, :16] + r[:, 16:]


def _split_bf16(w):
    w_hi = w.astype(jnp.bfloat16)
    w_lo = (w - w_hi.astype(jnp.float32)).astype(jnp.bfloat16)
    return w_hi, w_lo


def _msg(edge_attr, h_src, w1, b1, w2, b2):
    w1h, w1l = _split_bf16(w1)
    w2h, w2l = _split_bf16(w2)
    return pl.pallas_call(
        _msg_body,
        grid=(_EP // _EBLK,),
        in_specs=[
            pl.BlockSpec((_EBLK, D_EDGE), lambda i: (i, 0)),
            pl.BlockSpec((_EBLK, D_OUT), lambda i: (i, 0)),
            pl.BlockSpec((D_EDGE, EH), lambda i: (0, 0)),
            pl.BlockSpec((D_EDGE, EH), lambda i: (0, 0)),
            pl.BlockSpec((1, EH), lambda i: (0, 0)),
            pl.BlockSpec((EH, D_OUT * D_OUT), lambda i: (0, 0)),
            pl.BlockSpec((EH, D_OUT * D_OUT), lambda i: (0, 0)),
            pl.BlockSpec((1, D_OUT * D_OUT), lambda i: (0, 0)),
        ],
        out_specs=pl.BlockSpec((_EBLK, D_OUT), lambda i: (i, 0)),
        out_shape=jax.ShapeDtypeStruct((_EP, D_OUT), jnp.float32),
    )(edge_attr, h_src, w1h, w1l, b1.reshape(1, EH), w2h, w2l,
      b2.reshape(1, D_OUT * D_OUT))


def _segsum_body(gid_ref, nsh_ref, nse_ref, oh_ref, oe_ref):
    i = pl.program_id(0)

    @pl.when(i == 0)
    def _init():
        oh_ref[...] = jnp.zeros_like(oh_ref)
        oe_ref[...] = jnp.zeros_like(oe_ref)

    gid = gid_ref[0, 0, :]
    sel = (jax.lax.broadcasted_iota(jnp.int32, (B, _NBLK), 0)
           == gid[None, :]).astype(jnp.float32)
    oh_ref[...] += jnp.dot(sel, nsh_ref[...],
                           preferred_element_type=jnp.float32,
                           precision=jax.lax.Precision.HIGHEST)
    oe_ref[...] += jnp.dot(sel, nse_ref[...],
                           preferred_element_type=jnp.float32,
                           precision=jax.lax.Precision.HIGHEST)


def _segsum(gid, nsh, nse):
    return pl.pallas_call(
        _segsum_body,
        grid=(N // _NBLK,),
        in_specs=[
            pl.BlockSpec((1, 1, _NBLK), lambda i: (i, 0, 0)),
            pl.BlockSpec((_NBLK, D_OUT), lambda i: (i, 0)),
            pl.BlockSpec((_NBLK, D_EDGE), lambda i: (i, 0)),
        ],
        out_specs=[
            pl.BlockSpec((B, D_OUT), lambda i: (0, 0)),
            pl.BlockSpec((B, D_EDGE), lambda i: (0, 0)),
        ],
        out_shape=[
            jax.ShapeDtypeStruct((B, D_OUT), jnp.float32),
            jax.ShapeDtypeStruct((B, D_EDGE), jnp.float32),
        ],
    )(gid.reshape(N // _NBLK, 1, _NBLK), nsh, nse)


def _gru_body(p0_ref, p1_ref, cb_ref, h_ref, wihT_ref, whhT_ref, bih_ref,
              bhh_ref, o_ref):
    nf = jax.nn.relu(p0_ref[...] + p1_ref[...] + cb_ref[...])
    h = h_ref[...]
    gi = jnp.dot(nf, wihT_ref[...], preferred_element_type=jnp.float32, precision=jax.lax.Precision.HIGHEST) + bih_ref[...]
    gh = jnp.dot(h, whhT_ref[...], preferred_element_type=jnp.float32, precision=jax.lax.Precision.HIGHEST) + bhh_ref[...]
    i_r, i_z, i_n = gi[:, :D_OUT], gi[:, D_OUT:2 * D_OUT], gi[:, 2 * D_OUT:]
    h_r, h_z, h_n = gh[:, :D_OUT], gh[:, D_OUT:2 * D_OUT], gh[:, 2 * D_OUT:]
    r = jax.nn.sigmoid(i_r + h_r)
    z = jax.nn.sigmoid(i_z + h_z)
    n = jnp.tanh(i_n + r * h_n)
    o_ref[...] = (1.0 - z) * n + z * h


def _gru(p0, p1, conv_b, h, wihT, whhT, bih, bhh):
    g3 = 3 * D_OUT
    return pl.pallas_call(
        _gru_body,
        grid=(N // _NBLK,),
        in_specs=[
            pl.BlockSpec((_NBLK, D_OUT), lambda i: (i, 0)),
            pl.BlockSpec((_NBLK, D_OUT), lambda i: (i, 0)),
            pl.BlockSpec((1, D_OUT), lambda i: (0, 0)),
            pl.BlockSpec((_NBLK, D_OUT), lambda i: (i, 0)),
            pl.BlockSpec((D_OUT, g3), lambda i: (0, 0)),
            pl.BlockSpec((D_OUT, g3), lambda i: (0, 0)),
            pl.BlockSpec((1, g3), lambda i: (0, 0)),
            pl.BlockSpec((1, g3), lambda i: (0, 0)),
        ],
        out_specs=pl.BlockSpec((_NBLK, D_OUT), lambda i: (i, 0)),
        out_shape=jax.ShapeDtypeStruct((N, D_OUT), jnp.float32),
    )(p0, p1, conv_b.reshape(1, D_OUT), h, wihT, whhT,
      bih.reshape(1, g3), bhh.reshape(1, g3))


def _ffn_body(mol_ref, w1_ref, b1_ref, w2_ref, b2_ref, w3_ref, b3_ref, o_ref):
    h1 = jax.nn.relu(
        jnp.dot(mol_ref[...], w1_ref[...], preferred_element_type=jnp.float32, precision=jax.lax.Precision.HIGHEST)
        + b1_ref[...]
    )
    h2 = jax.nn.relu(
        jnp.dot(h1, w2_ref[...], preferred_element_type=jnp.float32, precision=jax.lax.Precision.HIGHEST) + b2_ref[...]
    )
    o_ref[...] = (
        jnp.dot(h2, w3_ref[...], preferred_element_type=jnp.float32, precision=jax.lax.Precision.HIGHEST) + b3_ref[...]
    )


def _ffn(mol, w1, b1, w2, b2, w3, b3):
    din = D_OUT + D_EDGE
    return pl.pallas_call(
        _ffn_body,
        in_specs=[
            pl.BlockSpec((B, din), lambda: (0, 0)),
            pl.BlockSpec((din, FFN_H), lambda: (0, 0)),
            pl.BlockSpec((1, FFN_H), lambda: (0, 0)),
            pl.BlockSpec((FFN_H, FFN_H), lambda: (0, 0)),
            pl.BlockSpec((1, FFN_H), lambda: (0, 0)),
            pl.BlockSpec((FFN_H, N_TASKS), lambda: (0, 0)),
            pl.BlockSpec((1, N_TASKS), lambda: (0, 0)),
        ],
        out_specs=pl.BlockSpec((B, N_TASKS), lambda: (0, 0)),
        out_shape=jax.ShapeDtypeStruct((B, N_TASKS), jnp.float32),
    )(mol, w1, b1.reshape(1, FFN_H), w2, b2.reshape(1, FFN_H), w3,
      b3.reshape(1, N_TASKS))


def kernel(x, edge_index, edge_attr, node_graph_ids, proj_W, proj_b, en_W1,
           en_b1, en_W2, en_b2, conv_b, gru_Wih, gru_Whh, gru_bih, gru_bhh,
           f1_W, f1_b, f2_W, f2_b, f3_W, f3_b):
    src = edge_index[0]
    dst = edge_index[1]
    wihT = gru_Wih.T
    whhT = gru_Whh.T

    pad = _EP - E
    src3 = jnp.concatenate(
        [src, jnp.zeros((pad,), jnp.int32)]).reshape(_NW, _NCH, _CHUNK)
    dst3 = jnp.concatenate(
        [dst, jnp.full((pad,), N, jnp.int32)]).reshape(_NW, _NCH, _CHUNK)
    ea_p = jnp.pad(edge_attr, ((0, pad), (0, 0)))

    h = _proj(x, proj_W, proj_b)

    for _ in range(STEPS):
        h_src = _sc_gather(h, src3)
        m = _msg(ea_p, h_src, en_W1, en_b1, en_W2, en_b2)
        parts = _sc_scatter_add(m, dst3)
        h = _gru(parts[0, :N], parts[1, :N], conv_b, h, wihT, whhT,
                 gru_bih, gru_bhh)

    hs = _sc_gather(h, src3)
    ph = _sc_scatter_add(hs, dst3)
    pe = _sc_scatter_add(ea_p, dst3)
    molH, molE = _segsum(node_graph_ids,
                         ph[0, :N] + ph[1, :N], pe[0, :N] + pe[1, :N])
    mol = jnp.concatenate([molH, molE], axis=1)
    return _ffn(mol, f1_W, f1_b, f2_W, f2_b, f3_W, f3_b)


# speedup vs baseline: 4.2661x; 1.0406x over previous
import functools

import jax
import jax.numpy as jnp
from jax import lax
from jax.experimental import pallas as pl
from jax.experimental.pallas import tpu as pltpu
from jax.experimental.pallas import tpu_sc as plsc

N = 10000
E = 160000
D_IN = 128
D_EDGE = 16
D_OUT = 16
EH = 128
STEPS = 3
B = 256
FFN_H = 300
N_TASKS = 8

_NC = 2
_NS = 16
_NW = _NC * _NS
_CHUNK = 128
_EP = 163840
_EPW = _EP // _NW
_NCH = _EPW // _CHUNK
_NP = 10240
_ZROWS = _NP // _NS


def _sc_mesh():
    return plsc.VectorSubcoreMesh(
        core_axis_name="c", subcore_axis_name="s",
        num_cores=_NC, num_subcores=_NS)


_SC_PARAMS = pltpu.CompilerParams(use_tc_tiling_on_sc=False)


_GCH = 1024
_NGCH = _EPW // _GCH


def _sc_gather(table, idx3):
    @functools.partial(
        pl.kernel,
        mesh=_sc_mesh(),
        out_type=jax.ShapeDtypeStruct((_EP, D_OUT), jnp.float32),
        scratch_types=[
            pltpu.VMEM((_NGCH, _GCH), jnp.int32),
            pltpu.VMEM((_EPW, D_OUT), jnp.float32),
            pltpu.SemaphoreType.DMA,
        ],
        compiler_params=_SC_PARAMS,
    )
    def k(table_hbm, idx_hbm, out_hbm, idx_v, rows_v, sem):
        wid = lax.axis_index("s") * _NC + lax.axis_index("c")
        pltpu.sync_copy(idx_hbm.at[wid], idx_v)
        copies = []
        for j in range(_NGCH):
            copies.append(
                pltpu.async_copy(
                    table_hbm.at[idx_v.at[j]],
                    rows_v.at[pl.ds(j * _GCH, _GCH)],
                    sem,
                )
            )
        for cp in copies:
            cp.wait()
        pltpu.sync_copy(rows_v, out_hbm.at[pl.ds(wid * _EPW, _EPW)])

    return k(table, idx3)


def _sc_scatter_add(vals, idx3):
    @functools.partial(
        pl.kernel,
        mesh=_sc_mesh(),
        out_type=jax.ShapeDtypeStruct((_NC, _NP, D_OUT), jnp.float32),
        scratch_types=[
            pltpu.VMEM((_NCH, _CHUNK), jnp.int32),
            pltpu.VMEM((_EPW, D_OUT), jnp.float32),
            pltpu.VMEM((_CHUNK, D_OUT), jnp.float32),
            pltpu.VMEM_SHARED((_NP, D_OUT), jnp.float32),
            pltpu.SemaphoreType.DMA,
        ],
        compiler_params=_SC_PARAMS,
    )
    def k(vals_hbm, idx_hbm, out_hbm, idx_v, vals_v, stage_v, acc_sh, sem):
        cid = lax.axis_index("c")
        sid = lax.axis_index("s")
        wid = sid * _NC + cid

        def _zero_row(i, _):
            stage_v[i] = jnp.zeros((D_OUT,), jnp.float32)
            return 0

        lax.fori_loop(0, _CHUNK, _zero_row, 0)
        for t in range(_ZROWS // _CHUNK):
            pltpu.sync_copy(
                stage_v, acc_sh.at[pl.ds(sid * _ZROWS + t * _CHUNK, _CHUNK)]
            )
        plsc.subcore_barrier()

        pltpu.sync_copy(idx_hbm.at[wid], idx_v)
        pltpu.sync_copy(vals_hbm.at[pl.ds(wid * _EPW, _EPW)], vals_v)
        for j in range(_NCH):
            pltpu.sync_copy(
                vals_v.at[pl.ds(j * _CHUNK, _CHUNK)],
                acc_sh.at[idx_v.at[j]],
                add=True,
            )
        plsc.subcore_barrier()

        for t in range(_ZROWS // _CHUNK):
            r0 = sid * _ZROWS + t * _CHUNK
            pltpu.sync_copy(acc_sh.at[pl.ds(r0, _CHUNK)], stage_v)
            pltpu.sync_copy(stage_v, out_hbm.at[cid].at[pl.ds(r0, _CHUNK)])

    return k(vals, idx3)


_NBLK = 2000
_EBLK = 8192


def _proj_body(x_ref, w_ref, b_ref, o_ref):
    o_ref[...] = jax.nn.relu(
        jnp.dot(x_ref[...], w_ref[...], preferred_element_type=jnp.float32, precision=jax.lax.Precision.HIGHEST)
        + b_ref[...]
    )


def _proj(x, w, b):
    return pl.pallas_call(
        _proj_body,
        grid=(N // _NBLK,),
        in_specs=[
            pl.BlockSpec((_NBLK, D_IN), lambda i: (i, 0)),
            pl.BlockSpec((D_IN, D_OUT), lambda i: (0, 0)),
            pl.BlockSpec((1, D_OUT), lambda i: (0, 0)),
        ],
        out_specs=pl.BlockSpec((_NBLK, D_OUT), lambda i: (i, 0)),
        out_shape=jax.ShapeDtypeStruct((N, D_OUT), jnp.float32),
    )(x, w, b.reshape(1, D_OUT))


def _dot3(x, w_hi, w_lo):
    x_hi = x.astype(jnp.bfloat16)
    x_lo = (x - x_hi.astype(jnp.float32)).astype(jnp.bfloat16)
    return (jnp.dot(x_hi, w_hi, preferred_element_type=jnp.float32)
            + (jnp.dot(x_hi, w_lo, preferred_element_type=jnp.float32)
               + jnp.dot(x_lo, w_hi, preferred_element_type=jnp.float32)))


def _msg_body(ea_ref, hs_ref, w1h_ref, w1l_ref, b1_ref, w2h_ref, w2l_ref,
              b2_ref, o_ref):
    a = jax.nn.relu(
        _dot3(ea_ref[...], w1h_ref[...], w1l_ref[...]) + b1_ref[...]
    )
    w = _dot3(a, w2h_ref[...], w2l_ref[...]) + b2_ref[...]
    lane = jax.lax.broadcasted_iota(
        jnp.int32, (_EBLK, D_OUT * D_OUT), 1) // D_OUT
    g = jnp.take_along_axis(hs_ref[...], lane, axis=1) * w
    r = g[:, :128] + g[:, 128:]
    r = r[:, :64] + r[:, 64:]
    r = r[:, :32] + r[:, 32:]
    o_ref[...] = r[:, :16] + r[:, 16:]


def _split_bf16(w):
    w_hi = w.astype(jnp.bfloat16)
    w_lo = (w - w_hi.astype(jnp.float32)).astype(jnp.bfloat16)
    return w_hi, w_lo


def _msg(edge_attr, h_src, w1, b1, w2, b2):
    w1h, w1l = _split_bf16(w1)
    w2h, w2l = _split_bf16(w2)
    return pl.pallas_call(
        _msg_body,
        grid=(_EP // _EBLK,),
        in_specs=[
            pl.BlockSpec((_EBLK, D_EDGE), lambda i: (i, 0)),
            pl.BlockSpec((_EBLK, D_OUT), lambda i: (i, 0)),
            pl.BlockSpec((D_EDGE, EH), lambda i: (0, 0)),
            pl.BlockSpec((D_EDGE, EH), lambda i: (0, 0)),
            pl.BlockSpec((1, EH), lambda i: (0, 0)),
            pl.BlockSpec((EH, D_OUT * D_OUT), lambda i: (0, 0)),
            pl.BlockSpec((EH, D_OUT * D_OUT), lambda i: (0, 0)),
            pl.BlockSpec((1, D_OUT * D_OUT), lambda i: (0, 0)),
        ],
        out_specs=pl.BlockSpec((_EBLK, D_OUT), lambda i: (i, 0)),
        out_shape=jax.ShapeDtypeStruct((_EP, D_OUT), jnp.float32),
    )(edge_attr, h_src, w1h, w1l, b1.reshape(1, EH), w2h, w2l,
      b2.reshape(1, D_OUT * D_OUT))


def _segsum_body(gid_ref, nsh_ref, nse_ref, oh_ref, oe_ref):
    i = pl.program_id(0)

    @pl.when(i == 0)
    def _init():
        oh_ref[...] = jnp.zeros_like(oh_ref)
        oe_ref[...] = jnp.zeros_like(oe_ref)

    gid = gid_ref[0, 0, :]
    sel = (jax.lax.broadcasted_iota(jnp.int32, (B, _NBLK), 0)
           == gid[None, :]).astype(jnp.float32)
    oh_ref[...] += jnp.dot(sel, nsh_ref[...],
                           preferred_element_type=jnp.float32,
                           precision=jax.lax.Precision.HIGHEST)
    oe_ref[...] += jnp.dot(sel, nse_ref[...],
                           preferred_element_type=jnp.float32,
                           precision=jax.lax.Precision.HIGHEST)


def _segsum(gid, nsh, nse):
    return pl.pallas_call(
        _segsum_body,
        grid=(N // _NBLK,),
        in_specs=[
            pl.BlockSpec((1, 1, _NBLK), lambda i: (i, 0, 0)),
            pl.BlockSpec((_NBLK, D_OUT), lambda i: (i, 0)),
            pl.BlockSpec((_NBLK, D_EDGE), lambda i: (i, 0)),
        ],
        out_specs=[
            pl.BlockSpec((B, D_OUT), lambda i: (0, 0)),
            pl.BlockSpec((B, D_EDGE), lambda i: (0, 0)),
        ],
        out_shape=[
            jax.ShapeDtypeStruct((B, D_OUT), jnp.float32),
            jax.ShapeDtypeStruct((B, D_EDGE), jnp.float32),
        ],
    )(gid.reshape(N // _NBLK, 1, _NBLK), nsh, nse)


def _gru_body(p0_ref, p1_ref, cb_ref, h_ref, wihT_ref, whhT_ref, bih_ref,
              bhh_ref, o_ref):
    nf = jax.nn.relu(p0_ref[0] + p1_ref[0] + cb_ref[...])
    h = h_ref[...]
    gi = jnp.dot(nf, wihT_ref[...], preferred_element_type=jnp.float32, precision=jax.lax.Precision.HIGHEST) + bih_ref[...]
    gh = jnp.dot(h, whhT_ref[...], preferred_element_type=jnp.float32, precision=jax.lax.Precision.HIGHEST) + bhh_ref[...]
    i_r, i_z, i_n = gi[:, :D_OUT], gi[:, D_OUT:2 * D_OUT], gi[:, 2 * D_OUT:]
    h_r, h_z, h_n = gh[:, :D_OUT], gh[:, D_OUT:2 * D_OUT], gh[:, 2 * D_OUT:]
    r = jax.nn.sigmoid(i_r + h_r)
    z = jax.nn.sigmoid(i_z + h_z)
    n = jnp.tanh(i_n + r * h_n)
    o_ref[...] = (1.0 - z) * n + z * h


def _gru(parts, conv_b, h, wihT, whhT, bih, bhh):
    g3 = 3 * D_OUT
    return pl.pallas_call(
        _gru_body,
        grid=(N // _NBLK,),
        in_specs=[
            pl.BlockSpec((1, _NBLK, D_OUT), lambda i: (0, i, 0)),
            pl.BlockSpec((1, _NBLK, D_OUT), lambda i: (1, i, 0)),
            pl.BlockSpec((1, D_OUT), lambda i: (0, 0)),
            pl.BlockSpec((_NBLK, D_OUT), lambda i: (i, 0)),
            pl.BlockSpec((D_OUT, g3), lambda i: (0, 0)),
            pl.BlockSpec((D_OUT, g3), lambda i: (0, 0)),
            pl.BlockSpec((1, g3), lambda i: (0, 0)),
            pl.BlockSpec((1, g3), lambda i: (0, 0)),
        ],
        out_specs=pl.BlockSpec((_NBLK, D_OUT), lambda i: (i, 0)),
        out_shape=jax.ShapeDtypeStruct((N, D_OUT), jnp.float32),
    )(parts, parts, conv_b.reshape(1, D_OUT), h, wihT, whhT,
      bih.reshape(1, g3), bhh.reshape(1, g3))


def _ffn_body(mol_ref, w1_ref, b1_ref, w2_ref, b2_ref, w3_ref, b3_ref, o_ref):
    h1 = jax.nn.relu(
        jnp.dot(mol_ref[...], w1_ref[...], preferred_element_type=jnp.float32, precision=jax.lax.Precision.HIGHEST)
        + b1_ref[...]
    )
    h2 = jax.nn.relu(
        jnp.dot(h1, w2_ref[...], preferred_element_type=jnp.float32, precision=jax.lax.Precision.HIGHEST) + b2_ref[...]
    )
    o_ref[...] = (
        jnp.dot(h2, w3_ref[...], preferred_element_type=jnp.float32, precision=jax.lax.Precision.HIGHEST) + b3_ref[...]
    )


def _ffn(mol, w1, b1, w2, b2, w3, b3):
    din = D_OUT + D_EDGE
    return pl.pallas_call(
        _ffn_body,
        in_specs=[
            pl.BlockSpec((B, din), lambda: (0, 0)),
            pl.BlockSpec((din, FFN_H), lambda: (0, 0)),
            pl.BlockSpec((1, FFN_H), lambda: (0, 0)),
            pl.BlockSpec((FFN_H, FFN_H), lambda: (0, 0)),
            pl.BlockSpec((1, FFN_H), lambda: (0, 0)),
            pl.BlockSpec((FFN_H, N_TASKS), lambda: (0, 0)),
            pl.BlockSpec((1, N_TASKS), lambda: (0, 0)),
        ],
        out_specs=pl.BlockSpec((B, N_TASKS), lambda: (0, 0)),
        out_shape=jax.ShapeDtypeStruct((B, N_TASKS), jnp.float32),
    )(mol, w1, b1.reshape(1, FFN_H), w2, b2.reshape(1, FFN_H), w3,
      b3.reshape(1, N_TASKS))


def kernel(x, edge_index, edge_attr, node_graph_ids, proj_W, proj_b, en_W1,
           en_b1, en_W2, en_b2, conv_b, gru_Wih, gru_Whh, gru_bih, gru_bhh,
           f1_W, f1_b, f2_W, f2_b, f3_W, f3_b):
    src = edge_index[0]
    dst = edge_index[1]
    wihT = gru_Wih.T
    whhT = gru_Whh.T

    pad = _EP - E
    src3 = jnp.concatenate(
        [src, jnp.zeros((pad,), jnp.int32)]).reshape(_NW, _NGCH, _GCH)
    dst3 = jnp.concatenate(
        [dst, jnp.full((pad,), N, jnp.int32)]).reshape(_NW, _NCH, _CHUNK)
    ea_p = jnp.pad(edge_attr, ((0, pad), (0, 0)))

    h = _proj(x, proj_W, proj_b)

    for _ in range(STEPS):
        h_src = _sc_gather(h, src3)
        m = _msg(ea_p, h_src, en_W1, en_b1, en_W2, en_b2)
        parts = _sc_scatter_add(m, dst3)
        h = _gru(parts, conv_b, h, wihT, whhT, gru_bih, gru_bhh)

    hs = _sc_gather(h, src3)
    ph = _sc_scatter_add(hs, dst3)
    pe = _sc_scatter_add(ea_p, dst3)
    molH, molE = _segsum(node_graph_ids,
                         ph[0, :N] + ph[1, :N], pe[0, :N] + pe[1, :N])
    mol = jnp.concatenate([molH, molE], axis=1)
    return _ffn(mol, f1_W, f1_b, f2_W, f2_b, f3_W, f3_b)
